# baseline (speedup 1.0000x reference)
import functools

import jax
import jax.numpy as jnp
from jax import lax
from jax.experimental import pallas as pl
from jax.experimental.pallas import tpu as pltpu
from jax.experimental.pallas import tpu_sc as plsc

NC = 2
NS = 16
NW = NC * NS

B = 4096 * 50
D = 128
B_W = B // NW
CHUNK = 128
NCHUNK = B_W // CHUNK
NBUF = 5
FIRE = 3

_mesh = plsc.VectorSubcoreMesh(core_axis_name="c", subcore_axis_name="s")


@functools.partial(
    pl.kernel,
    out_type=jax.ShapeDtypeStruct((B, D), jnp.float32),
    mesh=_mesh,
    scratch_types=[
        pltpu.VMEM((NCHUNK, CHUNK), jnp.int32),
        pltpu.VMEM((NBUF, CHUNK, D), jnp.float32),
        [pltpu.SemaphoreType.DMA] * NBUF,
        [pltpu.SemaphoreType.DMA] * NBUF,
    ],
)
def _gather_kernel(table_hbm, idx_hbm, out_hbm, idx_v, rows_v, gsems, osems):
    wid = lax.axis_index("s") * NC + lax.axis_index("c")
    base = wid * B_W
    pltpu.sync_copy(idx_hbm.at[wid], idx_v)

    def _fire_gather(chunk, b):
        pltpu.async_copy(
            table_hbm.at[idx_v.at[chunk]],
            rows_v.at[b],
            gsems[b],
        )

    def _fire_write(chunk, b):
        pltpu.async_copy(
            rows_v.at[b],
            out_hbm.at[pl.ds(base + chunk * CHUNK, CHUNK)],
            osems[b],
        )

    def _wait(sem, b):
        pltpu.make_async_copy(
            table_hbm.at[pl.ds(0, CHUNK)], rows_v.at[b], sem[b]
        ).wait()

    def _step(chunk, b, fire_m, wait_prev_write):
        _wait(gsems, b)
        _fire_write(chunk, b)
        m = chunk + FIRE
        if fire_m:
            bm = (b + FIRE) % NBUF
            if wait_prev_write:
                _wait(osems, bm)
            _fire_gather(m, bm)

    for j in range(FIRE):
        _fire_gather(j, j)
    for c in range(NBUF):
        _step(c, c, c + FIRE < NCHUNK, c + FIRE >= NBUF)

    @pl.loop(NBUF, NCHUNK - NBUF, step=NBUF)
    def _round(c0):
        for b in range(NBUF):
            _step(c0 + b, b, True, True)

    for c in range(NCHUNK - NBUF, NCHUNK):
        _step(c, c % NBUF, c + FIRE < NCHUNK, True)
    for b in range(NBUF):
        _wait(osems, b)


def kernel(x, table):
    idx = x.T.astype(jnp.int32).reshape(NW, NCHUNK, CHUNK)
    out = _gather_kernel(table, idx)
    return out.reshape(50, 4096, D).transpose(1, 0, 2)


# speedup vs baseline: 1.0275x; 1.0275x over previous
import functools

import jax
import jax.numpy as jnp
from jax import lax
from jax.experimental import pallas as pl
from jax.experimental.pallas import tpu as pltpu
from jax.experimental.pallas import tpu_sc as plsc

NC = 2
NS = 16
NW = NC * NS

B = 4096 * 50
D = 128
B_W = B // NW
CHUNK = 128
NCHUNK = B_W // CHUNK
NBUF = 5
FIRE = 3

_mesh = plsc.VectorSubcoreMesh(core_axis_name="c", subcore_axis_name="s")


@functools.partial(
    pl.kernel,
    out_type=jax.ShapeDtypeStruct((B, D), jnp.float32),
    mesh=_mesh,
    scratch_types=[
        pltpu.VMEM((NCHUNK, CHUNK), jnp.int32),
        pltpu.VMEM((NBUF, CHUNK, D), jnp.float32),
        [pltpu.SemaphoreType.DMA] * NBUF,
        [pltpu.SemaphoreType.DMA] * NBUF,
    ],
)
def _gather_kernel(table_hbm, idx_hbm, out_hbm, idx_v, rows_v, gsems, osems):
    wid = lax.axis_index("s") * NC + lax.axis_index("c")
    col = wid * CHUNK
    pltpu.sync_copy(idx_hbm.at[:, pl.ds(col, CHUNK)], idx_v)

    def _fire_gather(chunk, b):
        pltpu.async_copy(
            table_hbm.at[idx_v.at[chunk]],
            rows_v.at[b],
            gsems[b],
        )

    def _fire_write(chunk, b):
        pltpu.async_copy(
            rows_v.at[b],
            out_hbm.at[pl.ds(chunk * 4096 + col, CHUNK)],
            osems[b],
        )

    def _wait(sem, b):
        pltpu.make_async_copy(
            table_hbm.at[pl.ds(0, CHUNK)], rows_v.at[b], sem[b]
        ).wait()

    def _step(chunk, b, fire_m, wait_prev_write):
        _wait(gsems, b)
        _fire_write(chunk, b)
        m = chunk + FIRE
        if fire_m:
            bm = (b + FIRE) % NBUF
            if wait_prev_write:
                _wait(osems, bm)
            _fire_gather(m, bm)

    for j in range(FIRE):
        _fire_gather(j, j)
    for c in range(NBUF):
        _step(c, c, c + FIRE < NCHUNK, c + FIRE >= NBUF)

    @pl.loop(NBUF, NCHUNK - NBUF, step=NBUF)
    def _round(c0):
        for b in range(NBUF):
            _step(c0 + b, b, True, True)

    for c in range(NCHUNK - NBUF, NCHUNK):
        _step(c, c % NBUF, c + FIRE < NCHUNK, True)
    for b in range(NBUF):
        _wait(osems, b)


def kernel(x, table):
    idx = x.T.astype(jnp.int32)
    out = _gather_kernel(table, idx)
    return out.reshape(50, 4096, D).transpose(1, 0, 2)


# speedup vs baseline: 1.0289x; 1.0013x over previous
import functools

import jax
import jax.numpy as jnp
from jax import lax
from jax.experimental import pallas as pl
from jax.experimental.pallas import tpu as pltpu
from jax.experimental.pallas import tpu_sc as plsc

NC = 2
NS = 16
NW = NC * NS

B = 4096 * 50
D = 128
B_W = B // NW
CHUNK = 128
NCHUNK = B_W // CHUNK
NBUF = 5
FIRE = 4

_mesh = plsc.VectorSubcoreMesh(core_axis_name="c", subcore_axis_name="s")


@functools.partial(
    pl.kernel,
    out_type=jax.ShapeDtypeStruct((B, D), jnp.float32),
    mesh=_mesh,
    scratch_types=[
        pltpu.VMEM((NCHUNK, CHUNK), jnp.int32),
        pltpu.VMEM((NBUF, CHUNK, D), jnp.float32),
        [pltpu.SemaphoreType.DMA] * NBUF,
        [pltpu.SemaphoreType.DMA] * NBUF,
    ],
)
def _gather_kernel(table_hbm, idx_hbm, out_hbm, idx_v, rows_v, gsems, osems):
    wid = lax.axis_index("s") * NC + lax.axis_index("c")
    col = wid * CHUNK
    pltpu.sync_copy(idx_hbm.at[:, pl.ds(col, CHUNK)], idx_v)

    def _fire_gather(chunk, b):
        pltpu.async_copy(
            table_hbm.at[idx_v.at[chunk]],
            rows_v.at[b],
            gsems[b],
        )

    def _fire_write(chunk, b):
        pltpu.async_copy(
            rows_v.at[b],
            out_hbm.at[pl.ds(chunk * 4096 + col, CHUNK)],
            osems[b],
        )

    def _wait(sem, b):
        pltpu.make_async_copy(
            table_hbm.at[pl.ds(0, CHUNK)], rows_v.at[b], sem[b]
        ).wait()

    def _step(chunk, b, fire_m, wait_prev_write):
        _wait(gsems, b)
        _fire_write(chunk, b)
        m = chunk + FIRE
        if fire_m:
            bm = (b + FIRE) % NBUF
            if wait_prev_write:
                _wait(osems, bm)
            _fire_gather(m, bm)

    for j in range(FIRE):
        _fire_gather(j, j)
    for c in range(NBUF):
        _step(c, c, c + FIRE < NCHUNK, c + FIRE >= NBUF)

    @pl.loop(NBUF, NCHUNK - NBUF, step=NBUF)
    def _round(c0):
        for b in range(NBUF):
            _step(c0 + b, b, True, True)

    for c in range(NCHUNK - NBUF, NCHUNK):
        _step(c, c % NBUF, c + FIRE < NCHUNK, True)
    for b in range(NBUF):
        _wait(osems, b)


def kernel(x, table):
    idx = x.T.astype(jnp.int32)
    out = _gather_kernel(table, idx)
    return out.reshape(50, 4096, D).transpose(1, 0, 2)
